# Initial kernel scaffold; baseline (speedup 1.0000x reference)
import jax
import jax.numpy as jnp
from jax.experimental import pallas as pl


def kernel(x, W_route, b_route, W_noise, b_noise, W1, b1, W2, b2, noise):
    raise NotImplementedError("write your pallas kernel here")


# speedup vs baseline: 2.1960x; 2.1960x over previous
import functools

import jax
import jax.numpy as jnp
from jax import lax
from jax.experimental import pallas as pl
from jax.experimental.pallas import tpu as pltpu
from jax.experimental.pallas import tpu_sc as plsc

B, S, D, E, K, H = 2, 4096, 2048, 8, 2, 512
N = B * S
CAP = N * K // E
EC = E * CAP
NC, NS, L = 2, 16, 16

_BR = 1024


def _router_body(x_ref, w_ref, b_ref, noise_ref, t12_ref, g1_ref):
    lg = jnp.dot(x_ref[...], w_ref[...], preferred_element_type=jnp.float32)
    bias = b_ref[0:1, :]
    logits = lg[:, :E] + bias[:, :E]
    nlog = lg[:, E:] + bias[:, E:]
    sp = jnp.maximum(nlog, 0.0) + jnp.log1p(jnp.exp(-jnp.abs(nlog)))
    noisy = logits + noise_ref[...] * sp

    iota = lax.broadcasted_iota(jnp.int32, (_BR, E), 1)
    m1 = jnp.max(noisy, axis=1, keepdims=True)
    i1 = jnp.min(jnp.where(noisy == m1, iota, E), axis=1, keepdims=True)
    n2 = jnp.where(iota == i1, -jnp.inf, noisy)
    m2 = jnp.max(n2, axis=1, keepdims=True)
    i2 = jnp.min(jnp.where(n2 == m2, iota, E), axis=1, keepdims=True)
    r = jnp.exp(m2 - m1)
    g1 = 1.0 / (1.0 + r)
    t12_ref[...] = ((i1 << 3) | i2).reshape(_BR)
    g1_ref[...] = g1.reshape(_BR)


def _run_router(x2d, wcat, bcat, noise2d):
    return pl.pallas_call(
        _router_body,
        grid=(N // _BR,),
        in_specs=[
            pl.BlockSpec((_BR, D), lambda i: (i, 0)),
            pl.BlockSpec((D, 2 * E), lambda i: (0, 0)),
            pl.BlockSpec((8, 2 * E), lambda i: (0, 0)),
            pl.BlockSpec((_BR, E), lambda i: (i, 0)),
        ],
        out_specs=[
            pl.BlockSpec((_BR,), lambda i: (i,)),
            pl.BlockSpec((_BR,), lambda i: (i,)),
        ],
        out_shape=[
            jax.ShapeDtypeStruct((N,), jnp.int32),
            jax.ShapeDtypeStruct((N,), jnp.float32),
        ],
    )(x2d, wcat, bcat, noise2d)


_TT = N // NS
_TV = _TT // L
_SLOT_T = EC // NS

_disp_mesh = plsc.VectorSubcoreMesh(
    core_axis_name="c", subcore_axis_name="s", num_cores=1)


@functools.partial(
    pl.kernel,
    out_type=[
        jax.ShapeDtypeStruct((EC,), jnp.int32),
        jax.ShapeDtypeStruct((EC,), jnp.float32),
        jax.ShapeDtypeStruct((N,), jnp.int32),
        jax.ShapeDtypeStruct((N,), jnp.int32),
    ],
    mesh=_disp_mesh,
    scratch_types=[
        pltpu.VMEM((_TT,), jnp.int32),
        pltpu.VMEM((_TT,), jnp.float32),
        pltpu.VMEM((L,), jnp.int32),
        pltpu.VMEM((NS, L), jnp.int32),
        pltpu.VMEM((EC,), jnp.int32),
        pltpu.VMEM((EC,), jnp.float32),
        pltpu.VMEM((_TT,), jnp.int32),
        pltpu.VMEM((_TT,), jnp.int32),
        pltpu.VMEM((_SLOT_T,), jnp.int32),
        pltpu.VMEM((_SLOT_T,), jnp.float32),
        pltpu.VMEM((_SLOT_T,), jnp.int32),
        pltpu.VMEM((_SLOT_T,), jnp.float32),
        pltpu.VMEM_SHARED((NS, L), jnp.int32),
        pltpu.VMEM_SHARED((NS, EC), jnp.int32),
        pltpu.VMEM_SHARED((NS, EC), jnp.float32),
    ],
    compiler_params=pltpu.CompilerParams(needs_layout_passes=False),
)
def _dispatch(t12_hbm, g1_hbm, idx_hbm, gate_hbm, loc1_hbm, loc2_hbm,
              t12v, g1v, cntv, cnt_all, idxb, gateb, loc1b, loc2b,
              acc_i, acc_g, tmp_i, tmp_g, cnt_sh, idx_sh, gate_sh):
    sid = lax.axis_index("s")
    base = sid * _TT
    iota = lax.iota(jnp.int32, L)
    zero_i = jnp.zeros((L,), jnp.int32)
    zero_f = jnp.zeros((L,), jnp.float32)

    pltpu.sync_copy(t12_hbm.at[pl.ds(base, _TT)], t12v)
    pltpu.sync_copy(g1_hbm.at[pl.ds(base, _TT)], g1v)

    def _zero(i, _):
        idxb[pl.ds(i * L, L)] = zero_i
        gateb[pl.ds(i * L, L)] = zero_f
        return 0
    lax.fori_loop(0, EC // L, _zero, 0)

    def _count(i, cnt):
        t12x = t12v[pl.ds(i * L, L)]
        t1x = t12x >> 3
        t2x = t12x & 7
        for e in range(E):
            m = (t1x == e) | (t2x == e)
            c = jnp.sum(jnp.where(m, 1, 0))
            cnt = cnt + jnp.where(iota == e, c, 0)
        return cnt
    cnt = lax.fori_loop(0, _TV, _count, jnp.zeros((L,), jnp.int32))
    cntv[...] = cnt
    pltpu.sync_copy(cntv, cnt_sh.at[sid])
    plsc.subcore_barrier()
    pltpu.sync_copy(cnt_sh, cnt_all)

    offv = jnp.zeros((L,), jnp.int32)
    for w in range(NS):
        offv = offv + jnp.where(jnp.int32(w) < sid, cnt_all[w], 0)
    offs = [jnp.sum(jnp.where(iota == e, offv, 0)) for e in range(E)]

    def _scan(i, carry):
        offs = list(carry)
        sl = pl.ds(i * L, L)
        t12x = t12v[sl]
        t1x = t12x >> 3
        t2x = t12x & 7
        g1x = g1v[sl]
        g2x = 1.0 - g1x
        tok = base + i * L + iota
        l1 = jnp.full((L,), EC, jnp.int32)
        l2 = jnp.full((L,), EC, jnp.int32)
        for e in range(E):
            m1 = t1x == e
            m2 = t2x == e
            m = m1 | m2
            ones = jnp.where(m, 1, 0)
            cs = plsc.cumsum(ones)
            pos = offs[e] + cs - 1
            ok = m & (pos < CAP)
            dst = pos + e * CAP
            plsc.store_scatter(idxb, [dst], tok, mask=ok)
            gx = jnp.where(m1, g1x, g2x)
            plsc.store_scatter(gateb, [dst], gx, mask=ok)
            l1 = jnp.where(m1 & ok, dst, l1)
            l2 = jnp.where(m2 & ok, dst, l2)
            offs[e] = offs[e] + jnp.sum(ones)
        loc1b[sl] = l1
        loc2b[sl] = l2
        return tuple(offs)
    lax.fori_loop(0, _TV, _scan, tuple(offs))

    pltpu.sync_copy(loc1b, loc1_hbm.at[pl.ds(base, _TT)])
    pltpu.sync_copy(loc2b, loc2_hbm.at[pl.ds(base, _TT)])

    pltpu.sync_copy(idxb, idx_sh.at[sid])
    pltpu.sync_copy(gateb, gate_sh.at[sid])
    plsc.subcore_barrier()

    sbase = sid * _SLOT_T

    def _zacc(i, _):
        acc_i[pl.ds(i * L, L)] = zero_i
        acc_g[pl.ds(i * L, L)] = zero_f
        return 0
    lax.fori_loop(0, _SLOT_T // L, _zacc, 0)

    for w in range(NS):
        pltpu.sync_copy(idx_sh.at[w, pl.ds(sbase, _SLOT_T)], tmp_i)
        pltpu.sync_copy(gate_sh.at[w, pl.ds(sbase, _SLOT_T)], tmp_g)

        def _acc(i, _):
            sl = pl.ds(i * L, L)
            plsc.addupdate(acc_i.at[sl], tmp_i[sl])
            plsc.addupdate(acc_g.at[sl], tmp_g[sl])
            return 0
        lax.fori_loop(0, _SLOT_T // L, _acc, 0)

    pltpu.sync_copy(acc_i, idx_hbm.at[pl.ds(sbase, _SLOT_T)])
    pltpu.sync_copy(acc_g, gate_hbm.at[pl.ds(sbase, _SLOT_T)])


_GR = 32
_g_mesh = plsc.VectorSubcoreMesh(core_axis_name="c", subcore_axis_name="s")


@functools.partial(
    pl.kernel,
    out_type=jax.ShapeDtypeStruct((EC, D), jnp.float32),
    mesh=_g_mesh,
    scratch_types=[
        pltpu.VMEM((_GR,), jnp.int32),
        pltpu.VMEM((_GR, D), jnp.float32),
        pltpu.SemaphoreType.DMA,
    ],
    compiler_params=pltpu.CompilerParams(needs_layout_passes=False),
)
def _gather_rows(x_hbm, idx_hbm, out_hbm, idxv, rows, sem):
    wid = lax.axis_index("s") * NC + lax.axis_index("c")
    rpt = EC // (NC * NS)

    def _chunk(j, _):
        cb = wid * rpt + j * _GR
        pltpu.sync_copy(idx_hbm.at[pl.ds(cb, _GR)], idxv)
        pltpu.async_copy(x_hbm.at[idxv], rows, sem).wait()
        pltpu.sync_copy(rows, out_hbm.at[pl.ds(cb, _GR)])
        return 0
    lax.fori_loop(0, rpt // _GR, _chunk, 0)


_BM = 512
_NM = CAP // _BM
_CPAD = EC + _NM * _BM


def _ffn_body(x_ref, w1_ref, b1_ref, w2_ref, b2_ref, g_ref, c_ref):
    e = pl.program_id(0)

    @pl.when(e < E)
    def _():
        xb = x_ref[...]
        h = jnp.dot(xb, w1_ref[0], preferred_element_type=jnp.float32) + b1_ref[0]
        h = 0.5 * h * (1.0 + lax.erf(h * 0.7071067811865476))
        o = jnp.dot(h, w2_ref[0], preferred_element_type=jnp.float32) + b2_ref[0]
        g = g_ref[...].reshape(_BM, 1)
        c_ref[...] = (xb + o) * g

    @pl.when(e == E)
    def _():
        c_ref[...] = jnp.zeros((_BM, D), jnp.float32)


def _run_ffn(x_all, w1, b1r, w2, b2r, gate):
    ce = lambda e: jnp.minimum(e, E - 1)
    return pl.pallas_call(
        _ffn_body,
        grid=(E + 1, _NM),
        in_specs=[
            pl.BlockSpec((_BM, D), lambda e, m: (ce(e) * _NM + m, 0)),
            pl.BlockSpec((1, D, H), lambda e, m: (ce(e), 0, 0)),
            pl.BlockSpec((1, 1, H), lambda e, m: (ce(e), 0, 0)),
            pl.BlockSpec((1, H, D), lambda e, m: (ce(e), 0, 0)),
            pl.BlockSpec((1, 1, D), lambda e, m: (ce(e), 0, 0)),
            pl.BlockSpec((_BM,), lambda e, m: (ce(e) * _NM + m,)),
        ],
        out_specs=pl.BlockSpec((_BM, D), lambda e, m: (e * _NM + m, 0)),
        out_shape=jax.ShapeDtypeStruct((_CPAD, D), jnp.float32),
    )(x_all, w1, b1r, w2, b2r, gate)


_CT = 16


@functools.partial(
    pl.kernel,
    out_type=jax.ShapeDtypeStruct((N, D), jnp.float32),
    mesh=_g_mesh,
    scratch_types=[
        pltpu.VMEM((_CT,), jnp.int32),
        pltpu.VMEM((_CT,), jnp.int32),
        pltpu.VMEM((_CT, D), jnp.float32),
        pltpu.VMEM((_CT, D), jnp.float32),
        pltpu.SemaphoreType.DMA,
    ],
    compiler_params=pltpu.CompilerParams(needs_layout_passes=False),
)
def _combine(c_hbm, loc1_hbm, loc2_hbm, out_hbm, l1v, l2v, b1, b2, sem):
    wid = lax.axis_index("s") * NC + lax.axis_index("c")
    tpt = N // (NC * NS)

    def _chunk(j, _):
        cb = wid * tpt + j * _CT
        pltpu.sync_copy(loc1_hbm.at[pl.ds(cb, _CT)], l1v)
        pltpu.sync_copy(loc2_hbm.at[pl.ds(cb, _CT)], l2v)
        pltpu.async_copy(c_hbm.at[l1v], b1, sem).wait()
        pltpu.async_copy(c_hbm.at[l2v], b2, sem).wait()

        def _add(i, _):
            r = i // (D // L)
            k = i % (D // L)
            sl = pl.ds(k * L, L)
            plsc.addupdate(b1.at[r, sl], b2[r, sl])
            return 0
        lax.fori_loop(0, _CT * (D // L), _add, 0)
        pltpu.sync_copy(b1, out_hbm.at[pl.ds(cb, _CT)])
        return 0
    lax.fori_loop(0, tpt // _CT, _chunk, 0)


def kernel(x, W_route, b_route, W_noise, b_noise, W1, b1, W2, b2, noise):
    x2d = x.reshape(N, D)
    noise2d = noise.reshape(N, E)
    wcat = jnp.concatenate([W_route, W_noise], axis=1)
    bcat = jnp.tile(jnp.concatenate([b_route, b_noise])[None, :], (8, 1))

    t12, g1 = _run_router(x2d, wcat, bcat, noise2d)
    idx, gate, loc1, loc2 = _dispatch(t12, g1)
    x_all = _gather_rows(x2d, idx)
    c = _run_ffn(x_all, W1, b1.reshape(E, 1, H), W2, b2.reshape(E, 1, D), gate)
    out2d = _combine(c, loc1, loc2)
    return out2d.reshape(B, S, D)


# speedup vs baseline: 2.4901x; 1.1339x over previous
import functools

import jax
import jax.numpy as jnp
from jax import lax
from jax.experimental import pallas as pl
from jax.experimental.pallas import tpu as pltpu
from jax.experimental.pallas import tpu_sc as plsc

B, S, D, E, K, H = 2, 4096, 2048, 8, 2, 512
N = B * S
CAP = N * K // E
EC = E * CAP
NC, NS, L = 2, 16, 16

_BR = 1024


def _router_body(x_ref, w_ref, b_ref, noise_ref, t12_ref, g1_ref):
    lg = jnp.dot(x_ref[...], w_ref[...], preferred_element_type=jnp.float32)
    bias = b_ref[0:1, :]
    logits = lg[:, :E] + bias[:, :E]
    nlog = lg[:, E:] + bias[:, E:]
    sp = jnp.maximum(nlog, 0.0) + jnp.log1p(jnp.exp(-jnp.abs(nlog)))
    noisy = logits + noise_ref[...] * sp

    iota = lax.broadcasted_iota(jnp.int32, (_BR, E), 1)
    m1 = jnp.max(noisy, axis=1, keepdims=True)
    i1 = jnp.min(jnp.where(noisy == m1, iota, E), axis=1, keepdims=True)
    n2 = jnp.where(iota == i1, -jnp.inf, noisy)
    m2 = jnp.max(n2, axis=1, keepdims=True)
    i2 = jnp.min(jnp.where(n2 == m2, iota, E), axis=1, keepdims=True)
    r = jnp.exp(m2 - m1)
    g1 = 1.0 / (1.0 + r)
    t12_ref[...] = ((i1 << 3) | i2).reshape(_BR)
    g1_ref[...] = g1.reshape(_BR)


def _run_router(x2d, wcat, bcat, noise2d):
    return pl.pallas_call(
        _router_body,
        grid=(N // _BR,),
        in_specs=[
            pl.BlockSpec((_BR, D), lambda i: (i, 0)),
            pl.BlockSpec((D, 2 * E), lambda i: (0, 0)),
            pl.BlockSpec((8, 2 * E), lambda i: (0, 0)),
            pl.BlockSpec((_BR, E), lambda i: (i, 0)),
        ],
        out_specs=[
            pl.BlockSpec((_BR,), lambda i: (i,)),
            pl.BlockSpec((_BR,), lambda i: (i,)),
        ],
        out_shape=[
            jax.ShapeDtypeStruct((N,), jnp.int32),
            jax.ShapeDtypeStruct((N,), jnp.float32),
        ],
    )(x2d, wcat, bcat, noise2d)


_TT = N // NS
_TV = _TT // L
_SLOT_T = EC // NS

_disp_mesh = plsc.VectorSubcoreMesh(
    core_axis_name="c", subcore_axis_name="s", num_cores=1)


@functools.partial(
    pl.kernel,
    out_type=[
        jax.ShapeDtypeStruct((EC,), jnp.int32),
        jax.ShapeDtypeStruct((EC,), jnp.float32),
        jax.ShapeDtypeStruct((N,), jnp.int32),
        jax.ShapeDtypeStruct((N,), jnp.int32),
    ],
    mesh=_disp_mesh,
    scratch_types=[
        pltpu.VMEM((_TT,), jnp.int32),
        pltpu.VMEM((_TT,), jnp.float32),
        pltpu.VMEM((L,), jnp.int32),
        pltpu.VMEM((NS, L), jnp.int32),
        pltpu.VMEM((EC,), jnp.int32),
        pltpu.VMEM((EC,), jnp.float32),
        pltpu.VMEM((_TT,), jnp.int32),
        pltpu.VMEM((_TT,), jnp.int32),
        pltpu.VMEM((_SLOT_T,), jnp.int32),
        pltpu.VMEM((_SLOT_T,), jnp.float32),
        pltpu.VMEM((_SLOT_T,), jnp.int32),
        pltpu.VMEM((_SLOT_T,), jnp.float32),
        pltpu.VMEM_SHARED((NS, L), jnp.int32),
        pltpu.VMEM_SHARED((NS, EC), jnp.int32),
        pltpu.VMEM_SHARED((NS, EC), jnp.float32),
    ],
    compiler_params=pltpu.CompilerParams(needs_layout_passes=False),
)
def _dispatch(t12_hbm, g1_hbm, idx_hbm, gate_hbm, loc1_hbm, loc2_hbm,
              t12v, g1v, cntv, cnt_all, idxb, gateb, loc1b, loc2b,
              acc_i, acc_g, tmp_i, tmp_g, cnt_sh, idx_sh, gate_sh):
    sid = lax.axis_index("s")
    base = sid * _TT
    iota = lax.iota(jnp.int32, L)
    zero_i = jnp.zeros((L,), jnp.int32)
    zero_f = jnp.zeros((L,), jnp.float32)

    pltpu.sync_copy(t12_hbm.at[pl.ds(base, _TT)], t12v)
    pltpu.sync_copy(g1_hbm.at[pl.ds(base, _TT)], g1v)

    def _zero(i, _):
        idxb[pl.ds(i * L, L)] = zero_i
        gateb[pl.ds(i * L, L)] = zero_f
        return 0
    lax.fori_loop(0, EC // L, _zero, 0)

    def _count(i, cnt):
        t12x = t12v[pl.ds(i * L, L)]
        t1x = t12x >> 3
        t2x = t12x & 7
        for e in range(E):
            m = (t1x == e) | (t2x == e)
            c = jnp.sum(jnp.where(m, 1, 0))
            cnt = cnt + jnp.where(iota == e, c, 0)
        return cnt
    cnt = lax.fori_loop(0, _TV, _count, jnp.zeros((L,), jnp.int32))
    cntv[...] = cnt
    pltpu.sync_copy(cntv, cnt_sh.at[sid])
    plsc.subcore_barrier()
    pltpu.sync_copy(cnt_sh, cnt_all)

    offv = jnp.zeros((L,), jnp.int32)
    for w in range(NS):
        offv = offv + jnp.where(jnp.int32(w) < sid, cnt_all[w], 0)
    offs = [jnp.sum(jnp.where(iota == e, offv, 0)) for e in range(E)]

    def _scan(i, carry):
        offs = list(carry)
        sl = pl.ds(i * L, L)
        t12x = t12v[sl]
        t1x = t12x >> 3
        t2x = t12x & 7
        g1x = g1v[sl]
        g2x = 1.0 - g1x
        tok = base + i * L + iota
        l1 = jnp.full((L,), EC, jnp.int32)
        l2 = jnp.full((L,), EC, jnp.int32)
        for e in range(E):
            m1 = t1x == e
            m2 = t2x == e
            m = m1 | m2
            ones = jnp.where(m, 1, 0)
            cs = plsc.cumsum(ones)
            pos = offs[e] + cs - 1
            ok = m & (pos < CAP)
            dst = pos + e * CAP
            plsc.store_scatter(idxb, [dst], tok, mask=ok)
            gx = jnp.where(m1, g1x, g2x)
            plsc.store_scatter(gateb, [dst], gx, mask=ok)
            l1 = jnp.where(m1 & ok, dst, l1)
            l2 = jnp.where(m2 & ok, dst, l2)
            offs[e] = offs[e] + jnp.sum(ones)
        loc1b[sl] = l1
        loc2b[sl] = l2
        return tuple(offs)
    lax.fori_loop(0, _TV, _scan, tuple(offs))

    pltpu.sync_copy(loc1b, loc1_hbm.at[pl.ds(base, _TT)])
    pltpu.sync_copy(loc2b, loc2_hbm.at[pl.ds(base, _TT)])

    pltpu.sync_copy(idxb, idx_sh.at[sid])
    pltpu.sync_copy(gateb, gate_sh.at[sid])
    plsc.subcore_barrier()

    sbase = sid * _SLOT_T

    def _zacc(i, _):
        acc_i[pl.ds(i * L, L)] = zero_i
        acc_g[pl.ds(i * L, L)] = zero_f
        return 0
    lax.fori_loop(0, _SLOT_T // L, _zacc, 0)

    for w in range(NS):
        pltpu.sync_copy(idx_sh.at[w, pl.ds(sbase, _SLOT_T)], tmp_i)
        pltpu.sync_copy(gate_sh.at[w, pl.ds(sbase, _SLOT_T)], tmp_g)

        def _acc(i, _):
            sl = pl.ds(i * L, L)
            plsc.addupdate(acc_i.at[sl], tmp_i[sl])
            plsc.addupdate(acc_g.at[sl], tmp_g[sl])
            return 0
        lax.fori_loop(0, _SLOT_T // L, _acc, 0)

    pltpu.sync_copy(acc_i, idx_hbm.at[pl.ds(sbase, _SLOT_T)])
    pltpu.sync_copy(acc_g, gate_hbm.at[pl.ds(sbase, _SLOT_T)])


_GR = 16
_g_mesh = plsc.VectorSubcoreMesh(core_axis_name="c", subcore_axis_name="s")


@functools.partial(
    pl.kernel,
    out_type=jax.ShapeDtypeStruct((EC, D), jnp.float32),
    mesh=_g_mesh,
    scratch_types=[
        pltpu.VMEM((2, _GR), jnp.int32),
        pltpu.VMEM((2, _GR, D), jnp.float32),
        pltpu.SemaphoreType.DMA,
        pltpu.SemaphoreType.DMA,
    ],
    compiler_params=pltpu.CompilerParams(needs_layout_passes=False),
)
def _gather_rows(x_hbm, idx_hbm, out_hbm, idxv, rows, sem0, sem1):
    wid = lax.axis_index("s") * NC + lax.axis_index("c")
    rpt = EC // (NC * NS)
    nch = rpt // _GR
    base = wid * rpt
    sems = (sem0, sem1)

    for b in range(2):
        pltpu.sync_copy(idx_hbm.at[pl.ds(base + b * _GR, _GR)], idxv.at[b])
        pltpu.async_copy(x_hbm.at[idxv.at[b]], rows.at[b], sems[b])

    def _pair(m, _):
        for b in range(2):
            j = 2 * m + b
            pltpu.make_async_copy(x_hbm.at[idxv.at[b]], rows.at[b],
                                  sems[b]).wait()
            pltpu.sync_copy(rows.at[b], out_hbm.at[pl.ds(base + j * _GR, _GR)])

            @pl.when(j + 2 < nch)
            def _():
                pltpu.sync_copy(
                    idx_hbm.at[pl.ds(base + (j + 2) * _GR, _GR)], idxv.at[b])
                pltpu.async_copy(x_hbm.at[idxv.at[b]], rows.at[b], sems[b])
        return 0
    lax.fori_loop(0, nch // 2, _pair, 0)


_BM = 512
_NM = CAP // _BM
_CPAD = EC + _NM * _BM


def _ffn_body(x_ref, w1_ref, b1_ref, w2_ref, b2_ref, g_ref, c_ref):
    e = pl.program_id(0)

    @pl.when(e < E)
    def _():
        xb = x_ref[...]
        h = jnp.dot(xb, w1_ref[0], preferred_element_type=jnp.float32) + b1_ref[0]
        h = 0.5 * h * (1.0 + lax.erf(h * 0.7071067811865476))
        o = jnp.dot(h, w2_ref[0], preferred_element_type=jnp.float32) + b2_ref[0]
        g = g_ref[...].reshape(_BM, 1)
        c_ref[...] = (xb + o) * g

    @pl.when(e == E)
    def _():
        c_ref[...] = jnp.zeros((_BM, D), jnp.float32)


def _run_ffn(x_all, w1, b1r, w2, b2r, gate):
    ce = lambda e: jnp.minimum(e, E - 1)
    return pl.pallas_call(
        _ffn_body,
        grid=(E + 1, _NM),
        in_specs=[
            pl.BlockSpec((_BM, D), lambda e, m: (ce(e) * _NM + m, 0)),
            pl.BlockSpec((1, D, H), lambda e, m: (ce(e), 0, 0)),
            pl.BlockSpec((1, 1, H), lambda e, m: (ce(e), 0, 0)),
            pl.BlockSpec((1, H, D), lambda e, m: (ce(e), 0, 0)),
            pl.BlockSpec((1, 1, D), lambda e, m: (ce(e), 0, 0)),
            pl.BlockSpec((_BM,), lambda e, m: (ce(e) * _NM + m,)),
        ],
        out_specs=pl.BlockSpec((_BM, D), lambda e, m: (e * _NM + m, 0)),
        out_shape=jax.ShapeDtypeStruct((_CPAD, D), jnp.float32),
    )(x_all, w1, b1r, w2, b2r, gate)


_CT = 8


@functools.partial(
    pl.kernel,
    out_type=jax.ShapeDtypeStruct((N, D), jnp.float32),
    mesh=_g_mesh,
    scratch_types=[
        pltpu.VMEM((2, _CT), jnp.int32),
        pltpu.VMEM((2, _CT), jnp.int32),
        pltpu.VMEM((2, _CT, D), jnp.float32),
        pltpu.VMEM((2, _CT, D), jnp.float32),
        pltpu.SemaphoreType.DMA,
        pltpu.SemaphoreType.DMA,
    ],
    compiler_params=pltpu.CompilerParams(needs_layout_passes=False),
)
def _combine(c_hbm, loc1_hbm, loc2_hbm, out_hbm, l1v, l2v, b1, b2, sem0, sem1):
    wid = lax.axis_index("s") * NC + lax.axis_index("c")
    tpt = N // (NC * NS)
    nch = tpt // _CT
    base = wid * tpt
    sems = (sem0, sem1)

    def _fire(j, b):
        pltpu.sync_copy(loc1_hbm.at[pl.ds(base + j * _CT, _CT)], l1v.at[b])
        pltpu.sync_copy(loc2_hbm.at[pl.ds(base + j * _CT, _CT)], l2v.at[b])
        pltpu.async_copy(c_hbm.at[l1v.at[b]], b1.at[b], sems[b])
        pltpu.async_copy(c_hbm.at[l2v.at[b]], b2.at[b], sems[b])

    for b in range(2):
        _fire(b, b)

    def _pair(m, _):
        for b in range(2):
            j = 2 * m + b
            pltpu.make_async_copy(c_hbm.at[l1v.at[b]], b1.at[b],
                                  sems[b]).wait()
            pltpu.make_async_copy(c_hbm.at[l2v.at[b]], b2.at[b],
                                  sems[b]).wait()

            def _add(i, _):
                r = i // (D // L)
                k = i % (D // L)
                sl = pl.ds(k * L, L)
                plsc.addupdate(b1.at[b, r, sl], b2[b, r, sl])
                return 0
            lax.fori_loop(0, _CT * (D // L), _add, 0)
            pltpu.sync_copy(b1.at[b], out_hbm.at[pl.ds(base + j * _CT, _CT)])

            @pl.when(j + 2 < nch)
            def _():
                _fire(j + 2, b)
        return 0
    lax.fori_loop(0, nch // 2, _pair, 0)


def kernel(x, W_route, b_route, W_noise, b_noise, W1, b1, W2, b2, noise):
    x2d = x.reshape(N, D)
    noise2d = noise.reshape(N, E)
    wcat = jnp.concatenate([W_route, W_noise], axis=1)
    bcat = jnp.tile(jnp.concatenate([b_route, b_noise])[None, :], (8, 1))

    t12, g1 = _run_router(x2d, wcat, bcat, noise2d)
    idx, gate, loc1, loc2 = _dispatch(t12, g1)
    x_all = _gather_rows(x2d, idx)
    c = _run_ffn(x_all, W1, b1.reshape(E, 1, H), W2, b2.reshape(E, 1, D), gate)
    out2d = _combine(c, loc1, loc2)
    return out2d.reshape(B, S, D)


# speedup vs baseline: 2.8740x; 1.1542x over previous
import functools

import jax
import jax.numpy as jnp
from jax import lax
from jax.experimental import pallas as pl
from jax.experimental.pallas import tpu as pltpu
from jax.experimental.pallas import tpu_sc as plsc

B, S, D, E, K, H = 2, 4096, 2048, 8, 2, 512
N = B * S
CAP = N * K // E
EC = E * CAP
NC, NS, L = 2, 16, 16

_BR = 1024


_D2 = D // 2


def _pack16(v16):
    bits = lax.bitcast_convert_type(v16, jnp.uint16)
    lo = bits[:, :_D2].astype(jnp.int32)
    hi = bits[:, _D2:].astype(jnp.int32)
    return lo | (hi << 16)


def _unpack16(vi32):
    lo = vi32.astype(jnp.uint16)
    hi = lax.shift_right_logical(vi32, 16).astype(jnp.uint16)
    return jnp.concatenate(
        [lax.bitcast_convert_type(lo, jnp.bfloat16),
         lax.bitcast_convert_type(hi, jnp.bfloat16)], axis=1)


def _router_body(x_ref, w_ref, b_ref, noise_ref, t12_ref, g1_ref, xbf_ref):
    xbf_ref[...] = _pack16(x_ref[...].astype(jnp.bfloat16))
    lg = jnp.dot(x_ref[...], w_ref[...], preferred_element_type=jnp.float32)
    bias = b_ref[0:1, :]
    logits = lg[:, :E] + bias[:, :E]
    nlog = lg[:, E:] + bias[:, E:]
    sp = jnp.maximum(nlog, 0.0) + jnp.log1p(jnp.exp(-jnp.abs(nlog)))
    noisy = logits + noise_ref[...] * sp

    iota = lax.broadcasted_iota(jnp.int32, (_BR, E), 1)
    m1 = jnp.max(noisy, axis=1, keepdims=True)
    i1 = jnp.min(jnp.where(noisy == m1, iota, E), axis=1, keepdims=True)
    n2 = jnp.where(iota == i1, -jnp.inf, noisy)
    m2 = jnp.max(n2, axis=1, keepdims=True)
    i2 = jnp.min(jnp.where(n2 == m2, iota, E), axis=1, keepdims=True)
    r = jnp.exp(m2 - m1)
    g1 = 1.0 / (1.0 + r)
    t12_ref[...] = ((i1 << 3) | i2).reshape(_BR)
    g1_ref[...] = g1.reshape(_BR)


def _run_router(x2d, wcat, bcat, noise2d):
    return pl.pallas_call(
        _router_body,
        grid=(N // _BR,),
        in_specs=[
            pl.BlockSpec((_BR, D), lambda i: (i, 0)),
            pl.BlockSpec((D, 2 * E), lambda i: (0, 0)),
            pl.BlockSpec((8, 2 * E), lambda i: (0, 0)),
            pl.BlockSpec((_BR, E), lambda i: (i, 0)),
        ],
        out_specs=[
            pl.BlockSpec((_BR,), lambda i: (i,)),
            pl.BlockSpec((_BR,), lambda i: (i,)),
            pl.BlockSpec((_BR, _D2), lambda i: (i, 0)),
        ],
        out_shape=[
            jax.ShapeDtypeStruct((N,), jnp.int32),
            jax.ShapeDtypeStruct((N,), jnp.float32),
            jax.ShapeDtypeStruct((N, _D2), jnp.int32),
        ],
    )(x2d, wcat, bcat, noise2d)


_TT = N // NS
_TV = _TT // L
_SLOT_T = EC // NS

_disp_mesh = plsc.VectorSubcoreMesh(
    core_axis_name="c", subcore_axis_name="s", num_cores=1)


@functools.partial(
    pl.kernel,
    out_type=[
        jax.ShapeDtypeStruct((EC,), jnp.int32),
        jax.ShapeDtypeStruct((EC,), jnp.float32),
        jax.ShapeDtypeStruct((N,), jnp.int32),
        jax.ShapeDtypeStruct((N,), jnp.int32),
    ],
    mesh=_disp_mesh,
    scratch_types=[
        pltpu.VMEM((_TT,), jnp.int32),
        pltpu.VMEM((_TT,), jnp.float32),
        pltpu.VMEM((L,), jnp.int32),
        pltpu.VMEM((NS, L), jnp.int32),
        pltpu.VMEM((EC,), jnp.int32),
        pltpu.VMEM((EC,), jnp.float32),
        pltpu.VMEM((_TT,), jnp.int32),
        pltpu.VMEM((_TT,), jnp.int32),
        pltpu.VMEM((_SLOT_T,), jnp.int32),
        pltpu.VMEM((_SLOT_T,), jnp.float32),
        pltpu.VMEM((_SLOT_T,), jnp.int32),
        pltpu.VMEM((_SLOT_T,), jnp.float32),
        pltpu.VMEM_SHARED((NS, L), jnp.int32),
        pltpu.VMEM_SHARED((NS, EC), jnp.int32),
        pltpu.VMEM_SHARED((NS, EC), jnp.float32),
    ],
    compiler_params=pltpu.CompilerParams(needs_layout_passes=False),
)
def _dispatch(t12_hbm, g1_hbm, idx_hbm, gate_hbm, loc1_hbm, loc2_hbm,
              t12v, g1v, cntv, cnt_all, idxb, gateb, loc1b, loc2b,
              acc_i, acc_g, tmp_i, tmp_g, cnt_sh, idx_sh, gate_sh):
    sid = lax.axis_index("s")
    base = sid * _TT
    iota = lax.iota(jnp.int32, L)
    zero_i = jnp.zeros((L,), jnp.int32)
    zero_f = jnp.zeros((L,), jnp.float32)

    pltpu.sync_copy(t12_hbm.at[pl.ds(base, _TT)], t12v)
    pltpu.sync_copy(g1_hbm.at[pl.ds(base, _TT)], g1v)

    def _zero(i, _):
        idxb[pl.ds(i * L, L)] = zero_i
        gateb[pl.ds(i * L, L)] = zero_f
        return 0
    lax.fori_loop(0, EC // L, _zero, 0)

    def _count(i, cnt):
        t12x = t12v[pl.ds(i * L, L)]
        t1x = t12x >> 3
        t2x = t12x & 7
        for e in range(E):
            m = (t1x == e) | (t2x == e)
            c = jnp.sum(jnp.where(m, 1, 0))
            cnt = cnt + jnp.where(iota == e, c, 0)
        return cnt
    cnt = lax.fori_loop(0, _TV, _count, jnp.zeros((L,), jnp.int32))
    cntv[...] = cnt
    pltpu.sync_copy(cntv, cnt_sh.at[sid])
    plsc.subcore_barrier()
    pltpu.sync_copy(cnt_sh, cnt_all)

    offv = jnp.zeros((L,), jnp.int32)
    for w in range(NS):
        offv = offv + jnp.where(jnp.int32(w) < sid, cnt_all[w], 0)
    offs = [jnp.sum(jnp.where(iota == e, offv, 0)) for e in range(E)]

    def _scan(i, carry):
        offs = list(carry)
        sl = pl.ds(i * L, L)
        t12x = t12v[sl]
        t1x = t12x >> 3
        t2x = t12x & 7
        g1x = g1v[sl]
        g2x = 1.0 - g1x
        tok = base + i * L + iota
        l1 = jnp.full((L,), EC, jnp.int32)
        l2 = jnp.full((L,), EC, jnp.int32)
        for e in range(E):
            m1 = t1x == e
            m2 = t2x == e
            m = m1 | m2
            ones = jnp.where(m, 1, 0)
            cs = plsc.cumsum(ones)
            pos = offs[e] + cs - 1
            ok = m & (pos < CAP)
            dst = pos + e * CAP
            plsc.store_scatter(idxb, [dst], tok, mask=ok)
            gx = jnp.where(m1, g1x, g2x)
            plsc.store_scatter(gateb, [dst], gx, mask=ok)
            l1 = jnp.where(m1 & ok, dst, l1)
            l2 = jnp.where(m2 & ok, dst, l2)
            offs[e] = offs[e] + jnp.sum(ones)
        loc1b[sl] = l1
        loc2b[sl] = l2
        return tuple(offs)
    lax.fori_loop(0, _TV, _scan, tuple(offs))

    pltpu.sync_copy(loc1b, loc1_hbm.at[pl.ds(base, _TT)])
    pltpu.sync_copy(loc2b, loc2_hbm.at[pl.ds(base, _TT)])

    pltpu.sync_copy(idxb, idx_sh.at[sid])
    pltpu.sync_copy(gateb, gate_sh.at[sid])
    plsc.subcore_barrier()

    sbase = sid * _SLOT_T

    def _zacc(i, _):
        acc_i[pl.ds(i * L, L)] = zero_i
        acc_g[pl.ds(i * L, L)] = zero_f
        return 0
    lax.fori_loop(0, _SLOT_T // L, _zacc, 0)

    for w in range(NS):
        pltpu.sync_copy(idx_sh.at[w, pl.ds(sbase, _SLOT_T)], tmp_i)
        pltpu.sync_copy(gate_sh.at[w, pl.ds(sbase, _SLOT_T)], tmp_g)

        def _acc(i, _):
            sl = pl.ds(i * L, L)
            plsc.addupdate(acc_i.at[sl], tmp_i[sl])
            plsc.addupdate(acc_g.at[sl], tmp_g[sl])
            return 0
        lax.fori_loop(0, _SLOT_T // L, _acc, 0)

    pltpu.sync_copy(acc_i, idx_hbm.at[pl.ds(sbase, _SLOT_T)])
    pltpu.sync_copy(acc_g, gate_hbm.at[pl.ds(sbase, _SLOT_T)])


_GR = 32
_g_mesh = plsc.VectorSubcoreMesh(core_axis_name="c", subcore_axis_name="s")


@functools.partial(
    pl.kernel,
    out_type=jax.ShapeDtypeStruct((EC, _D2), jnp.int32),
    mesh=_g_mesh,
    scratch_types=[
        pltpu.VMEM((2, _GR), jnp.int32),
        pltpu.VMEM((2, _GR, _D2), jnp.int32),
        pltpu.SemaphoreType.DMA,
        pltpu.SemaphoreType.DMA,
    ],
    compiler_params=pltpu.CompilerParams(needs_layout_passes=False),
)
def _gather_rows(x_hbm, idx_hbm, out_hbm, idxv, rows, sem0, sem1):
    wid = lax.axis_index("s") * NC + lax.axis_index("c")
    rpt = EC // (NC * NS)
    nch = rpt // _GR
    base = wid * rpt
    sems = (sem0, sem1)

    for b in range(2):
        pltpu.sync_copy(idx_hbm.at[pl.ds(base + b * _GR, _GR)], idxv.at[b])
        pltpu.async_copy(x_hbm.at[idxv.at[b]], rows.at[b], sems[b])

    def _pair(m, _):
        for b in range(2):
            j = 2 * m + b
            pltpu.make_async_copy(x_hbm.at[idxv.at[b]], rows.at[b],
                                  sems[b]).wait()
            pltpu.sync_copy(rows.at[b], out_hbm.at[pl.ds(base + j * _GR, _GR)])

            @pl.when(j + 2 < nch)
            def _():
                pltpu.sync_copy(
                    idx_hbm.at[pl.ds(base + (j + 2) * _GR, _GR)], idxv.at[b])
                pltpu.async_copy(x_hbm.at[idxv.at[b]], rows.at[b], sems[b])
        return 0
    lax.fori_loop(0, nch // 2, _pair, 0)


_BM = 512
_NM = CAP // _BM
_CPAD = EC + _NM * _BM


def _ffn_body(x_ref, w1_ref, b1_ref, w2_ref, b2_ref, g_ref, c_ref):
    e = pl.program_id(0)

    @pl.when(e < E)
    def _():
        xb = _unpack16(x_ref[...]).astype(jnp.float32)
        h = jnp.dot(xb, w1_ref[0], preferred_element_type=jnp.float32) + b1_ref[0]
        h = 0.5 * h * (1.0 + lax.erf(h * 0.7071067811865476))
        o = jnp.dot(h, w2_ref[0], preferred_element_type=jnp.float32) + b2_ref[0]
        g = g_ref[...].reshape(_BM, 1)
        c_ref[...] = _pack16(((xb + o) * g).astype(jnp.bfloat16))

    @pl.when(e == E)
    def _():
        c_ref[...] = jnp.zeros((_BM, _D2), jnp.int32)


def _run_ffn(x_all, w1, b1r, w2, b2r, gate):
    ce = lambda e: jnp.minimum(e, E - 1)
    return pl.pallas_call(
        _ffn_body,
        grid=(E + 1, _NM),
        in_specs=[
            pl.BlockSpec((_BM, _D2), lambda e, m: (ce(e) * _NM + m, 0)),
            pl.BlockSpec((1, D, H), lambda e, m: (ce(e), 0, 0)),
            pl.BlockSpec((1, 1, H), lambda e, m: (ce(e), 0, 0)),
            pl.BlockSpec((1, H, D), lambda e, m: (ce(e), 0, 0)),
            pl.BlockSpec((1, 1, D), lambda e, m: (ce(e), 0, 0)),
            pl.BlockSpec((_BM,), lambda e, m: (ce(e) * _NM + m,)),
        ],
        out_specs=pl.BlockSpec((_BM, _D2), lambda e, m: (e * _NM + m, 0)),
        out_shape=jax.ShapeDtypeStruct((_CPAD, _D2), jnp.int32),
    )(x_all, w1, b1r, w2, b2r, gate)


_CT = 16


@functools.partial(
    pl.kernel,
    out_type=jax.ShapeDtypeStruct((N, _D2), jnp.int32),
    mesh=_g_mesh,
    scratch_types=[
        pltpu.VMEM((2, _CT), jnp.int32),
        pltpu.VMEM((2, _CT), jnp.int32),
        pltpu.VMEM((2, _CT, _D2), jnp.int32),
        pltpu.VMEM((2, _CT, _D2), jnp.int32),
        pltpu.SemaphoreType.DMA,
        pltpu.SemaphoreType.DMA,
    ],
    compiler_params=pltpu.CompilerParams(needs_layout_passes=False),
)
def _combine(c_hbm, loc1_hbm, loc2_hbm, out_hbm, l1v, l2v, b1, b2, sem0, sem1):
    wid = lax.axis_index("s") * NC + lax.axis_index("c")
    tpt = N // (NC * NS)
    nch = tpt // _CT
    base = wid * tpt
    sems = (sem0, sem1)

    def _fire(j, b):
        pltpu.sync_copy(loc1_hbm.at[pl.ds(base + j * _CT, _CT)], l1v.at[b])
        pltpu.sync_copy(loc2_hbm.at[pl.ds(base + j * _CT, _CT)], l2v.at[b])
        pltpu.async_copy(c_hbm.at[l1v.at[b]], b1.at[b], sems[b])
        pltpu.async_copy(c_hbm.at[l2v.at[b]], b2.at[b], sems[b])

    for b in range(2):
        _fire(b, b)

    def _pair(m, _):
        for b in range(2):
            j = 2 * m + b
            pltpu.make_async_copy(c_hbm.at[l1v.at[b]], b1.at[b],
                                  sems[b]).wait()
            pltpu.make_async_copy(c_hbm.at[l2v.at[b]], b2.at[b],
                                  sems[b]).wait()

            def _add(i, _):
                r = i // (_D2 // L)
                k = i % (_D2 // L)
                sl = pl.ds(k * L, L)
                v1 = plsc.bitcast(b1[b, r, sl], jnp.bfloat16)
                v2 = plsc.bitcast(b2[b, r, sl], jnp.bfloat16)
                b1[b, r, sl] = plsc.bitcast(v1 + v2, jnp.int32)
                return 0
            lax.fori_loop(0, _CT * (_D2 // L), _add, 0)
            pltpu.sync_copy(b1.at[b], out_hbm.at[pl.ds(base + j * _CT, _CT)])

            @pl.when(j + 2 < nch)
            def _():
                _fire(j + 2, b)
        return 0
    lax.fori_loop(0, nch // 2, _pair, 0)


def kernel(x, W_route, b_route, W_noise, b_noise, W1, b1, W2, b2, noise):
    x2d = x.reshape(N, D)
    noise2d = noise.reshape(N, E)
    wcat = jnp.concatenate([W_route, W_noise], axis=1)
    bcat = jnp.tile(jnp.concatenate([b_route, b_noise])[None, :], (8, 1))

    t12, g1, xbf = _run_router(x2d, wcat, bcat, noise2d)
    idx, gate, loc1, loc2 = _dispatch(t12, g1)
    x_all = _gather_rows(xbf, idx)
    c = _run_ffn(x_all, W1, b1.reshape(E, 1, H), W2, b2.reshape(E, 1, D), gate)
    out2d = _combine(c, loc1, loc2)
    lo = out2d.astype(jnp.uint16)
    hi = lax.shift_right_logical(out2d, 16).astype(jnp.uint16)
    out16 = jnp.concatenate(
        [lax.bitcast_convert_type(lo, jnp.bfloat16),
         lax.bitcast_convert_type(hi, jnp.bfloat16)], axis=1)
    return out16.astype(jnp.float32).reshape(B, S, D)


# speedup vs baseline: 2.9567x; 1.0288x over previous
import functools

import jax
import jax.numpy as jnp
from jax import lax
from jax.experimental import pallas as pl
from jax.experimental.pallas import tpu as pltpu
from jax.experimental.pallas import tpu_sc as plsc

B, S, D, E, K, H = 2, 4096, 2048, 8, 2, 512
N = B * S
CAP = N * K // E
EC = E * CAP
NC, NS, L = 2, 16, 16

_BR = 1024


_D2 = D // 2


def _pack16(v16):
    bits = lax.bitcast_convert_type(v16, jnp.uint16)
    lo = bits[:, :_D2].astype(jnp.int32)
    hi = bits[:, _D2:].astype(jnp.int32)
    return lo | (hi << 16)


def _unpack16(vi32):
    lo = vi32.astype(jnp.uint16)
    hi = lax.shift_right_logical(vi32, 16).astype(jnp.uint16)
    return jnp.concatenate(
        [lax.bitcast_convert_type(lo, jnp.bfloat16),
         lax.bitcast_convert_type(hi, jnp.bfloat16)], axis=1)


def _router_body(x_ref, w_ref, b_ref, noise_ref, t12_ref, g1_ref, xbf_ref):
    xbf_ref[...] = _pack16(x_ref[...].astype(jnp.bfloat16))
    lg = jnp.dot(x_ref[...], w_ref[...], preferred_element_type=jnp.float32)
    bias = b_ref[0:1, :]
    logits = lg[:, :E] + bias[:, :E]
    nlog = lg[:, E:] + bias[:, E:]
    sp = jnp.maximum(nlog, 0.0) + jnp.log1p(jnp.exp(-jnp.abs(nlog)))
    noisy = logits + noise_ref[...] * sp

    iota = lax.broadcasted_iota(jnp.int32, (_BR, E), 1)
    m1 = jnp.max(noisy, axis=1, keepdims=True)
    i1 = jnp.min(jnp.where(noisy == m1, iota, E), axis=1, keepdims=True)
    n2 = jnp.where(iota == i1, -jnp.inf, noisy)
    m2 = jnp.max(n2, axis=1, keepdims=True)
    i2 = jnp.min(jnp.where(n2 == m2, iota, E), axis=1, keepdims=True)
    r = jnp.exp(m2 - m1)
    g1 = 1.0 / (1.0 + r)
    t12_ref[...] = ((i1 << 3) | i2).reshape(_BR)
    g1_ref[...] = g1.reshape(_BR)


def _run_router(x2d, wcat, bcat, noise2d):
    return pl.pallas_call(
        _router_body,
        grid=(N // _BR,),
        in_specs=[
            pl.BlockSpec((_BR, D), lambda i: (i, 0)),
            pl.BlockSpec((D, 2 * E), lambda i: (0, 0)),
            pl.BlockSpec((8, 2 * E), lambda i: (0, 0)),
            pl.BlockSpec((_BR, E), lambda i: (i, 0)),
        ],
        out_specs=[
            pl.BlockSpec((_BR,), lambda i: (i,)),
            pl.BlockSpec((_BR,), lambda i: (i,)),
            pl.BlockSpec((_BR, _D2), lambda i: (i, 0)),
        ],
        out_shape=[
            jax.ShapeDtypeStruct((N,), jnp.int32),
            jax.ShapeDtypeStruct((N,), jnp.float32),
            jax.ShapeDtypeStruct((N, _D2), jnp.int32),
        ],
    )(x2d, wcat, bcat, noise2d)


_TT = N // NS
_TV = _TT // L
_SLOT_T = EC // NS

_disp_mesh = plsc.VectorSubcoreMesh(
    core_axis_name="c", subcore_axis_name="s", num_cores=1)


@functools.partial(
    pl.kernel,
    out_type=[
        jax.ShapeDtypeStruct((EC,), jnp.int32),
        jax.ShapeDtypeStruct((EC,), jnp.float32),
        jax.ShapeDtypeStruct((N,), jnp.int32),
        jax.ShapeDtypeStruct((N,), jnp.int32),
    ],
    mesh=_disp_mesh,
    scratch_types=[
        pltpu.VMEM((_TT,), jnp.int32),
        pltpu.VMEM((_TT,), jnp.float32),
        pltpu.VMEM((L,), jnp.int32),
        pltpu.VMEM((NS, L), jnp.int32),
        pltpu.VMEM((EC,), jnp.int32),
        pltpu.VMEM((EC,), jnp.float32),
        pltpu.VMEM((_TT,), jnp.int32),
        pltpu.VMEM((_TT,), jnp.int32),
        pltpu.VMEM((_SLOT_T,), jnp.int32),
        pltpu.VMEM((_SLOT_T,), jnp.float32),
        pltpu.VMEM((_SLOT_T,), jnp.int32),
        pltpu.VMEM((_SLOT_T,), jnp.float32),
        pltpu.VMEM_SHARED((NS, L), jnp.int32),
        pltpu.VMEM_SHARED((NS, EC), jnp.int32),
        pltpu.VMEM_SHARED((NS, EC), jnp.float32),
    ],
    compiler_params=pltpu.CompilerParams(needs_layout_passes=False),
)
def _dispatch(t12_hbm, g1_hbm, idx_hbm, gate_hbm, loc1_hbm, loc2_hbm,
              t12v, g1v, cntv, cnt_all, idxb, gateb, loc1b, loc2b,
              acc_i, acc_g, tmp_i, tmp_g, cnt_sh, idx_sh, gate_sh):
    sid = lax.axis_index("s")
    base = sid * _TT
    iota = lax.iota(jnp.int32, L)
    zero_i = jnp.zeros((L,), jnp.int32)
    zero_f = jnp.zeros((L,), jnp.float32)

    pltpu.sync_copy(t12_hbm.at[pl.ds(base, _TT)], t12v)
    pltpu.sync_copy(g1_hbm.at[pl.ds(base, _TT)], g1v)

    def _zero(i, _):
        idxb[pl.ds(i * L, L)] = zero_i
        gateb[pl.ds(i * L, L)] = zero_f
        return 0
    lax.fori_loop(0, EC // L, _zero, 0)

    def _count(i, cnt):
        t12x = t12v[pl.ds(i * L, L)]
        t1x = t12x >> 3
        t2x = t12x & 7
        for e in range(E):
            m = (t1x == e) | (t2x == e)
            c = jnp.sum(jnp.where(m, 1, 0))
            cnt = cnt + jnp.where(iota == e, c, 0)
        return cnt
    cnt = lax.fori_loop(0, _TV, _count, jnp.zeros((L,), jnp.int32))
    cntv[...] = cnt
    pltpu.sync_copy(cntv, cnt_sh.at[sid])
    plsc.subcore_barrier()
    pltpu.sync_copy(cnt_sh, cnt_all)

    offv = jnp.zeros((L,), jnp.int32)
    for w in range(NS):
        offv = offv + jnp.where(jnp.int32(w) < sid, cnt_all[w], 0)
    offs = [jnp.sum(jnp.where(iota == e, offv, 0)) for e in range(E)]

    def _scan(i, carry):
        offs = list(carry)
        sl = pl.ds(i * L, L)
        t12x = t12v[sl]
        t1x = t12x >> 3
        t2x = t12x & 7
        g1x = g1v[sl]
        g2x = 1.0 - g1x
        tok = base + i * L + iota
        l1 = jnp.full((L,), EC, jnp.int32)
        l2 = jnp.full((L,), EC, jnp.int32)
        for e in range(E):
            m1 = t1x == e
            m2 = t2x == e
            m = m1 | m2
            ones = jnp.where(m, 1, 0)
            cs = plsc.cumsum(ones)
            pos = offs[e] + cs - 1
            ok = m & (pos < CAP)
            dst = pos + e * CAP
            plsc.store_scatter(idxb, [dst], tok, mask=ok)
            gx = jnp.where(m1, g1x, g2x)
            plsc.store_scatter(gateb, [dst], gx, mask=ok)
            l1 = jnp.where(m1 & ok, dst, l1)
            l2 = jnp.where(m2 & ok, dst, l2)
            offs[e] = offs[e] + jnp.sum(ones)
        loc1b[sl] = l1
        loc2b[sl] = l2
        return tuple(offs)
    lax.fori_loop(0, _TV, _scan, tuple(offs))

    pltpu.sync_copy(loc1b, loc1_hbm.at[pl.ds(base, _TT)])
    pltpu.sync_copy(loc2b, loc2_hbm.at[pl.ds(base, _TT)])

    pltpu.sync_copy(idxb, idx_sh.at[sid])
    pltpu.sync_copy(gateb, gate_sh.at[sid])
    plsc.subcore_barrier()

    sbase = sid * _SLOT_T

    def _zacc(i, _):
        acc_i[pl.ds(i * L, L)] = zero_i
        acc_g[pl.ds(i * L, L)] = zero_f
        return 0
    lax.fori_loop(0, _SLOT_T // L, _zacc, 0)

    for w in range(NS):
        pltpu.sync_copy(idx_sh.at[w, pl.ds(sbase, _SLOT_T)], tmp_i)
        pltpu.sync_copy(gate_sh.at[w, pl.ds(sbase, _SLOT_T)], tmp_g)

        def _acc(i, _):
            sl = pl.ds(i * L, L)
            plsc.addupdate(acc_i.at[sl], tmp_i[sl])
            plsc.addupdate(acc_g.at[sl], tmp_g[sl])
            return 0
        lax.fori_loop(0, _SLOT_T // L, _acc, 0)

    pltpu.sync_copy(acc_i, idx_hbm.at[pl.ds(sbase, _SLOT_T)])
    pltpu.sync_copy(acc_g, gate_hbm.at[pl.ds(sbase, _SLOT_T)])


_GR = 32
_g_mesh = plsc.VectorSubcoreMesh(core_axis_name="c", subcore_axis_name="s")


@functools.partial(
    pl.kernel,
    out_type=jax.ShapeDtypeStruct((EC, _D2), jnp.int32),
    mesh=_g_mesh,
    scratch_types=[
        pltpu.VMEM((2, _GR), jnp.int32),
        pltpu.VMEM((2, _GR, _D2), jnp.int32),
        pltpu.SemaphoreType.DMA,
        pltpu.SemaphoreType.DMA,
    ],
    compiler_params=pltpu.CompilerParams(needs_layout_passes=False),
)
def _gather_rows(x_hbm, idx_hbm, out_hbm, idxv, rows, sem0, sem1):
    wid = lax.axis_index("s") * NC + lax.axis_index("c")
    rpt = EC // (NC * NS)
    nch = rpt // _GR
    base = wid * rpt
    sems = (sem0, sem1)

    for b in range(2):
        pltpu.sync_copy(idx_hbm.at[pl.ds(base + b * _GR, _GR)], idxv.at[b])
        pltpu.async_copy(x_hbm.at[idxv.at[b]], rows.at[b], sems[b])

    def _pair(m, _):
        for b in range(2):
            j = 2 * m + b
            pltpu.make_async_copy(x_hbm.at[idxv.at[b]], rows.at[b],
                                  sems[b]).wait()
            pltpu.sync_copy(rows.at[b], out_hbm.at[pl.ds(base + j * _GR, _GR)])

            @pl.when(j + 2 < nch)
            def _():
                pltpu.sync_copy(
                    idx_hbm.at[pl.ds(base + (j + 2) * _GR, _GR)], idxv.at[b])
                pltpu.async_copy(x_hbm.at[idxv.at[b]], rows.at[b], sems[b])
        return 0
    lax.fori_loop(0, nch // 2, _pair, 0)


_BM = 512
_NM = CAP // _BM
_CPAD = EC + _NM * _BM


def _ffn_body(x_ref, w1_ref, b1_ref, w2_ref, b2_ref, g_ref, c_ref):
    e = pl.program_id(0)

    @pl.when(e < E)
    def _():
        xb16 = _unpack16(x_ref[...])
        h = jnp.dot(xb16, w1_ref[0].astype(jnp.bfloat16),
                    preferred_element_type=jnp.float32) + b1_ref[0]
        h = 0.5 * h * (1.0 + lax.erf(h * 0.7071067811865476))
        o = jnp.dot(h.astype(jnp.bfloat16), w2_ref[0].astype(jnp.bfloat16),
                    preferred_element_type=jnp.float32) + b2_ref[0]
        g = g_ref[...].reshape(_BM, 1)
        c_ref[...] = _pack16(((xb16.astype(jnp.float32) + o) * g)
                             .astype(jnp.bfloat16))

    @pl.when(e == E)
    def _():
        c_ref[...] = jnp.zeros((_BM, _D2), jnp.int32)


def _run_ffn(x_all, w1, b1r, w2, b2r, gate):
    ce = lambda e: jnp.minimum(e, E - 1)
    return pl.pallas_call(
        _ffn_body,
        grid=(E + 1, _NM),
        in_specs=[
            pl.BlockSpec((_BM, _D2), lambda e, m: (ce(e) * _NM + m, 0)),
            pl.BlockSpec((1, D, H), lambda e, m: (ce(e), 0, 0)),
            pl.BlockSpec((1, 1, H), lambda e, m: (ce(e), 0, 0)),
            pl.BlockSpec((1, H, D), lambda e, m: (ce(e), 0, 0)),
            pl.BlockSpec((1, 1, D), lambda e, m: (ce(e), 0, 0)),
            pl.BlockSpec((_BM,), lambda e, m: (ce(e) * _NM + m,)),
        ],
        out_specs=pl.BlockSpec((_BM, _D2), lambda e, m: (e * _NM + m, 0)),
        out_shape=jax.ShapeDtypeStruct((_CPAD, _D2), jnp.int32),
    )(x_all, w1, b1r, w2, b2r, gate)


_CT = 8
_MSK = jnp.int32(-65536)


@functools.partial(
    pl.kernel,
    out_type=jax.ShapeDtypeStruct((N, D), jnp.float32),
    mesh=_g_mesh,
    scratch_types=[
        pltpu.VMEM((2, _CT), jnp.int32),
        pltpu.VMEM((2, _CT), jnp.int32),
        pltpu.VMEM((2, _CT, _D2), jnp.int32),
        pltpu.VMEM((2, _CT, _D2), jnp.int32),
        pltpu.VMEM((2, _CT, D), jnp.float32),
        pltpu.SemaphoreType.DMA,
        pltpu.SemaphoreType.DMA,
    ],
    compiler_params=pltpu.CompilerParams(needs_layout_passes=False),
)
def _combine(c_hbm, loc1_hbm, loc2_hbm, out_hbm, l1v, l2v, b1, b2, bo,
             sem0, sem1):
    wid = lax.axis_index("s") * NC + lax.axis_index("c")
    tpt = N // (NC * NS)
    nch = tpt // _CT
    base = wid * tpt
    sems = (sem0, sem1)

    def _fire(j, b):
        pltpu.sync_copy(loc1_hbm.at[pl.ds(base + j * _CT, _CT)], l1v.at[b])
        pltpu.sync_copy(loc2_hbm.at[pl.ds(base + j * _CT, _CT)], l2v.at[b])
        pltpu.async_copy(c_hbm.at[l1v.at[b]], b1.at[b], sems[b])
        pltpu.async_copy(c_hbm.at[l2v.at[b]], b2.at[b], sems[b])

    for b in range(2):
        _fire(b, b)

    def _pair(m, _):
        for b in range(2):
            j = 2 * m + b
            pltpu.make_async_copy(c_hbm.at[l1v.at[b]], b1.at[b],
                                  sems[b]).wait()
            pltpu.make_async_copy(c_hbm.at[l2v.at[b]], b2.at[b],
                                  sems[b]).wait()

            def _add(i, _):
                r = i // (_D2 // L)
                k = i % (_D2 // L)
                sl = pl.ds(k * L, L)
                v1 = b1[b, r, sl]
                v2 = b2[b, r, sl]
                lo = (plsc.bitcast(v1 << 16, jnp.float32)
                      + plsc.bitcast(v2 << 16, jnp.float32))
                hi = (plsc.bitcast(v1 & _MSK, jnp.float32)
                      + plsc.bitcast(v2 & _MSK, jnp.float32))
                bo[b, r, sl] = lo
                bo[b, r, pl.ds(_D2 + k * L, L)] = hi
                return 0
            lax.fori_loop(0, _CT * (_D2 // L), _add, 0)
            pltpu.sync_copy(bo.at[b], out_hbm.at[pl.ds(base + j * _CT, _CT)])

            @pl.when(j + 2 < nch)
            def _():
                _fire(j + 2, b)
        return 0
    lax.fori_loop(0, nch // 2, _pair, 0)


def kernel(x, W_route, b_route, W_noise, b_noise, W1, b1, W2, b2, noise):
    x2d = x.reshape(N, D)
    noise2d = noise.reshape(N, E)
    wcat = jnp.concatenate([W_route, W_noise], axis=1)
    bcat = jnp.tile(jnp.concatenate([b_route, b_noise])[None, :], (8, 1))

    t12, g1, xbf = _run_router(x2d, wcat, bcat, noise2d)
    idx, gate, loc1, loc2 = _dispatch(t12, g1)
    x_all = _gather_rows(xbf, idx)
    c = _run_ffn(x_all, W1, b1.reshape(E, 1, H), W2, b2.reshape(E, 1, D), gate)
    out2d = _combine(c, loc1, loc2)
    return out2d.reshape(B, S, D)


# speedup vs baseline: 3.5525x; 1.2015x over previous
import functools

import jax
import jax.numpy as jnp
from jax import lax
from jax.experimental import pallas as pl
from jax.experimental.pallas import tpu as pltpu
from jax.experimental.pallas import tpu_sc as plsc

B, S, D, E, K, H = 2, 4096, 2048, 8, 2, 512
N = B * S
CAP = N * K // E
EC = E * CAP
NC, NS, L = 2, 16, 16

_BR = 1024


_D2 = D // 2


def _pack16(v16):
    bits = lax.bitcast_convert_type(v16, jnp.uint16)
    lo = bits[:, :_D2].astype(jnp.int32)
    hi = bits[:, _D2:].astype(jnp.int32)
    return lo | (hi << 16)


def _unpack16(vi32):
    lo = vi32.astype(jnp.uint16)
    hi = lax.shift_right_logical(vi32, 16).astype(jnp.uint16)
    return jnp.concatenate(
        [lax.bitcast_convert_type(lo, jnp.bfloat16),
         lax.bitcast_convert_type(hi, jnp.bfloat16)], axis=1)


def _router_body(x_ref, w_ref, b_ref, noise_ref, t12_ref, g1_ref, xbf_ref):
    xbf_ref[...] = _pack16(x_ref[...].astype(jnp.bfloat16))
    lg = jnp.dot(x_ref[...], w_ref[...], preferred_element_type=jnp.float32)
    bias = b_ref[0:1, :]
    logits = lg[:, :E] + bias[:, :E]
    nlog = lg[:, E:] + bias[:, E:]
    sp = jnp.maximum(nlog, 0.0) + jnp.log1p(jnp.exp(-jnp.abs(nlog)))
    noisy = logits + noise_ref[...] * sp

    iota = lax.broadcasted_iota(jnp.int32, (_BR, E), 1)
    m1 = jnp.max(noisy, axis=1, keepdims=True)
    i1 = jnp.min(jnp.where(noisy == m1, iota, E), axis=1, keepdims=True)
    n2 = jnp.where(iota == i1, -jnp.inf, noisy)
    m2 = jnp.max(n2, axis=1, keepdims=True)
    i2 = jnp.min(jnp.where(n2 == m2, iota, E), axis=1, keepdims=True)
    r = jnp.exp(m2 - m1)
    g1 = 1.0 / (1.0 + r)
    t12_ref[...] = ((i1 << 3) | i2).reshape(_BR)
    g1_ref[...] = g1.reshape(_BR)


def _run_router(x2d, wcat, bcat, noise2d):
    return pl.pallas_call(
        _router_body,
        grid=(N // _BR,),
        in_specs=[
            pl.BlockSpec((_BR, D), lambda i: (i, 0)),
            pl.BlockSpec((D, 2 * E), lambda i: (0, 0)),
            pl.BlockSpec((8, 2 * E), lambda i: (0, 0)),
            pl.BlockSpec((_BR, E), lambda i: (i, 0)),
        ],
        out_specs=[
            pl.BlockSpec((_BR,), lambda i: (i,)),
            pl.BlockSpec((_BR,), lambda i: (i,)),
            pl.BlockSpec((_BR, _D2), lambda i: (i, 0)),
        ],
        out_shape=[
            jax.ShapeDtypeStruct((N,), jnp.int32),
            jax.ShapeDtypeStruct((N,), jnp.float32),
            jax.ShapeDtypeStruct((N, _D2), jnp.int32),
        ],
    )(x2d, wcat, bcat, noise2d)


_TT = N // NS
_TV = _TT // L
_SLOT_T = EC // NS

_disp_mesh = plsc.VectorSubcoreMesh(
    core_axis_name="c", subcore_axis_name="s", num_cores=1)


@functools.partial(
    pl.kernel,
    out_type=[
        jax.ShapeDtypeStruct((EC,), jnp.int32),
        jax.ShapeDtypeStruct((EC,), jnp.float32),
        jax.ShapeDtypeStruct((N,), jnp.int32),
        jax.ShapeDtypeStruct((N,), jnp.int32),
    ],
    mesh=_disp_mesh,
    scratch_types=[
        pltpu.VMEM((_TT,), jnp.int32),
        pltpu.VMEM((_TT,), jnp.float32),
        pltpu.VMEM((L,), jnp.int32),
        pltpu.VMEM((NS, L), jnp.int32),
        pltpu.VMEM((EC,), jnp.int32),
        pltpu.VMEM((EC,), jnp.float32),
        pltpu.VMEM((_TT,), jnp.int32),
        pltpu.VMEM((_TT,), jnp.int32),
        pltpu.VMEM((_SLOT_T,), jnp.int32),
        pltpu.VMEM((_SLOT_T,), jnp.float32),
        pltpu.VMEM((_SLOT_T,), jnp.int32),
        pltpu.VMEM((_SLOT_T,), jnp.float32),
        pltpu.VMEM_SHARED((NS, L), jnp.int32),
        pltpu.VMEM_SHARED((NS, EC), jnp.int32),
        pltpu.VMEM_SHARED((NS, EC), jnp.float32),
    ],
    compiler_params=pltpu.CompilerParams(needs_layout_passes=False),
)
def _dispatch(t12_hbm, g1_hbm, idx_hbm, gate_hbm, loc1_hbm, loc2_hbm,
              t12v, g1v, cntv, cnt_all, idxb, gateb, loc1b, loc2b,
              acc_i, acc_g, tmp_i, tmp_g, cnt_sh, idx_sh, gate_sh):
    sid = lax.axis_index("s")
    base = sid * _TT
    iota = lax.iota(jnp.int32, L)
    zero_i = jnp.zeros((L,), jnp.int32)
    zero_f = jnp.zeros((L,), jnp.float32)

    pltpu.sync_copy(t12_hbm.at[pl.ds(base, _TT)], t12v)
    pltpu.sync_copy(g1_hbm.at[pl.ds(base, _TT)], g1v)

    def _zero(i, _):
        idxb[pl.ds(i * L, L)] = zero_i
        gateb[pl.ds(i * L, L)] = zero_f
        return 0
    lax.fori_loop(0, EC // L, _zero, 0)

    def _count(i, cnt):
        t12x = t12v[pl.ds(i * L, L)]
        t1x = t12x >> 3
        t2x = t12x & 7
        for e in range(E):
            m = (t1x == e) | (t2x == e)
            c = jnp.sum(jnp.where(m, 1, 0))
            cnt = cnt + jnp.where(iota == e, c, 0)
        return cnt
    cnt = lax.fori_loop(0, _TV, _count, jnp.zeros((L,), jnp.int32))
    cntv[...] = cnt
    pltpu.sync_copy(cntv, cnt_sh.at[sid])
    plsc.subcore_barrier()
    pltpu.sync_copy(cnt_sh, cnt_all)

    offv = jnp.zeros((L,), jnp.int32)
    for w in range(NS):
        offv = offv + jnp.where(jnp.int32(w) < sid, cnt_all[w], 0)
    offs = [jnp.sum(jnp.where(iota == e, offv, 0)) for e in range(E)]

    def _scan(i, carry):
        offs = list(carry)
        sl = pl.ds(i * L, L)
        t12x = t12v[sl]
        t1x = t12x >> 3
        t2x = t12x & 7
        g1x = g1v[sl]
        g2x = 1.0 - g1x
        tok = base + i * L + iota
        l1 = jnp.full((L,), EC, jnp.int32)
        l2 = jnp.full((L,), EC, jnp.int32)
        for e in range(E):
            m1 = t1x == e
            m2 = t2x == e
            m = m1 | m2
            ones = jnp.where(m, 1, 0)
            cs = plsc.cumsum(ones)
            pos = offs[e] + cs - 1
            ok = m & (pos < CAP)
            dst = pos + e * CAP
            plsc.store_scatter(idxb, [dst], tok, mask=ok)
            gx = jnp.where(m1, g1x, g2x)
            plsc.store_scatter(gateb, [dst], gx, mask=ok)
            l1 = jnp.where(m1 & ok, dst, l1)
            l2 = jnp.where(m2 & ok, dst, l2)
            offs[e] = offs[e] + jnp.sum(ones)
        loc1b[sl] = l1
        loc2b[sl] = l2
        return tuple(offs)
    lax.fori_loop(0, _TV, _scan, tuple(offs))

    pltpu.sync_copy(loc1b, loc1_hbm.at[pl.ds(base, _TT)])
    pltpu.sync_copy(loc2b, loc2_hbm.at[pl.ds(base, _TT)])

    pltpu.sync_copy(idxb, idx_sh.at[sid])
    pltpu.sync_copy(gateb, gate_sh.at[sid])
    plsc.subcore_barrier()

    sbase = sid * _SLOT_T

    def _zacc(i, _):
        acc_i[pl.ds(i * L, L)] = zero_i
        acc_g[pl.ds(i * L, L)] = zero_f
        return 0
    lax.fori_loop(0, _SLOT_T // L, _zacc, 0)

    for w in range(NS):
        pltpu.sync_copy(idx_sh.at[w, pl.ds(sbase, _SLOT_T)], tmp_i)
        pltpu.sync_copy(gate_sh.at[w, pl.ds(sbase, _SLOT_T)], tmp_g)

        def _acc(i, _):
            sl = pl.ds(i * L, L)
            plsc.addupdate(acc_i.at[sl], tmp_i[sl])
            plsc.addupdate(acc_g.at[sl], tmp_g[sl])
            return 0
        lax.fori_loop(0, _SLOT_T // L, _acc, 0)

    pltpu.sync_copy(acc_i, idx_hbm.at[pl.ds(sbase, _SLOT_T)])
    pltpu.sync_copy(acc_g, gate_hbm.at[pl.ds(sbase, _SLOT_T)])


_GR = 32
_g_mesh = plsc.VectorSubcoreMesh(core_axis_name="c", subcore_axis_name="s")


@functools.partial(
    pl.kernel,
    out_type=jax.ShapeDtypeStruct((EC, _D2), jnp.int32),
    mesh=_g_mesh,
    scratch_types=[
        pltpu.VMEM((2, _GR), jnp.int32),
        pltpu.VMEM((2, _GR, _D2), jnp.int32),
        pltpu.SemaphoreType.DMA,
        pltpu.SemaphoreType.DMA,
    ],
    compiler_params=pltpu.CompilerParams(needs_layout_passes=False),
)
def _gather_rows(x_hbm, idx_hbm, out_hbm, idxv, rows, sem0, sem1):
    wid = lax.axis_index("s") * NC + lax.axis_index("c")
    rpt = EC // (NC * NS)
    nch = rpt // _GR
    base = wid * rpt
    sems = (sem0, sem1)

    for b in range(2):
        pltpu.sync_copy(idx_hbm.at[pl.ds(base + b * _GR, _GR)], idxv.at[b])
        pltpu.async_copy(x_hbm.at[idxv.at[b]], rows.at[b], sems[b])

    def _pair(m, _):
        for b in range(2):
            j = 2 * m + b
            pltpu.make_async_copy(x_hbm.at[idxv.at[b]], rows.at[b],
                                  sems[b]).wait()
            pltpu.sync_copy(rows.at[b], out_hbm.at[pl.ds(base + j * _GR, _GR)])

            @pl.when(j + 2 < nch)
            def _():
                pltpu.sync_copy(
                    idx_hbm.at[pl.ds(base + (j + 2) * _GR, _GR)], idxv.at[b])
                pltpu.async_copy(x_hbm.at[idxv.at[b]], rows.at[b], sems[b])
        return 0
    lax.fori_loop(0, nch // 2, _pair, 0)


_BM = 512
_NM = CAP // _BM
_CPAD = EC + _NM * _BM


def _ffn_body(x_ref, w1_ref, b1_ref, w2_ref, b2_ref, g_ref, c_ref):
    e = pl.program_id(0)

    @pl.when(e < E)
    def _():
        xb16 = _unpack16(x_ref[...])
        h = jnp.dot(xb16, w1_ref[0].astype(jnp.bfloat16),
                    preferred_element_type=jnp.float32) + b1_ref[0]
        h = 0.5 * h * (1.0 + lax.erf(h * 0.7071067811865476))
        o = jnp.dot(h.astype(jnp.bfloat16), w2_ref[0].astype(jnp.bfloat16),
                    preferred_element_type=jnp.float32) + b2_ref[0]
        g = g_ref[...].reshape(_BM, 1)
        c_ref[...] = _pack16(((xb16.astype(jnp.float32) + o) * g)
                             .astype(jnp.bfloat16))

    @pl.when(e == E)
    def _():
        c_ref[...] = jnp.zeros((_BM, _D2), jnp.int32)


def _run_ffn(x_all, w1, b1r, w2, b2r, gate):
    ce = lambda e: jnp.minimum(e, E - 1)
    return pl.pallas_call(
        _ffn_body,
        grid=(E + 1, _NM),
        in_specs=[
            pl.BlockSpec((_BM, _D2), lambda e, m: (ce(e) * _NM + m, 0)),
            pl.BlockSpec((1, D, H), lambda e, m: (ce(e), 0, 0)),
            pl.BlockSpec((1, 1, H), lambda e, m: (ce(e), 0, 0)),
            pl.BlockSpec((1, H, D), lambda e, m: (ce(e), 0, 0)),
            pl.BlockSpec((1, 1, D), lambda e, m: (ce(e), 0, 0)),
            pl.BlockSpec((_BM,), lambda e, m: (ce(e) * _NM + m,)),
        ],
        out_specs=pl.BlockSpec((_BM, _D2), lambda e, m: (e * _NM + m, 0)),
        out_shape=jax.ShapeDtypeStruct((_CPAD, _D2), jnp.int32),
    )(x_all, w1, b1r, w2, b2r, gate)


_CT = 8
_MSK = jnp.int32(-65536)


@functools.partial(
    pl.kernel,
    out_type=jax.ShapeDtypeStruct((N, D), jnp.float32),
    mesh=_g_mesh,
    scratch_types=[
        pltpu.VMEM((2, _CT), jnp.int32),
        pltpu.VMEM((2, _CT), jnp.int32),
        pltpu.VMEM((2, _CT, _D2), jnp.int32),
        pltpu.VMEM((2, _CT, _D2), jnp.int32),
        pltpu.VMEM((2, _CT, D), jnp.float32),
        pltpu.SemaphoreType.DMA,
        pltpu.SemaphoreType.DMA,
        pltpu.SemaphoreType.DMA,
        pltpu.SemaphoreType.DMA,
    ],
    compiler_params=pltpu.CompilerParams(needs_layout_passes=False),
)
def _combine(c_hbm, loc1_hbm, loc2_hbm, out_hbm, l1v, l2v, b1, b2, bo,
             sem0, sem1, so0, so1):
    wid = lax.axis_index("s") * NC + lax.axis_index("c")
    tpt = N // (NC * NS)
    nch = tpt // _CT
    base = wid * tpt
    sems = (sem0, sem1)
    souts = (so0, so1)

    def _fire(j, b):
        pltpu.sync_copy(loc1_hbm.at[pl.ds(base + j * _CT, _CT)], l1v.at[b])
        pltpu.sync_copy(loc2_hbm.at[pl.ds(base + j * _CT, _CT)], l2v.at[b])
        pltpu.async_copy(c_hbm.at[l1v.at[b]], b1.at[b], sems[b])
        pltpu.async_copy(c_hbm.at[l2v.at[b]], b2.at[b], sems[b])

    for b in range(2):
        _fire(b, b)

    def _pair(m, _):
        for b in range(2):
            j = 2 * m + b
            pltpu.make_async_copy(c_hbm.at[l1v.at[b]], b1.at[b],
                                  sems[b]).wait()
            pltpu.make_async_copy(c_hbm.at[l2v.at[b]], b2.at[b],
                                  sems[b]).wait()

            @pl.when(m > 0)
            def _():
                pltpu.make_async_copy(
                    bo.at[b], out_hbm.at[pl.ds(base, _CT)], souts[b]).wait()

            def _add(r, _):
                for k in range(_D2 // L):
                    sl = pl.ds(k * L, L)
                    v1 = b1[b, r, sl]
                    v2 = b2[b, r, sl]
                    lo = (plsc.bitcast(v1 << 16, jnp.float32)
                          + plsc.bitcast(v2 << 16, jnp.float32))
                    hi = (plsc.bitcast(v1 & _MSK, jnp.float32)
                          + plsc.bitcast(v2 & _MSK, jnp.float32))
                    bo[b, r, sl] = lo
                    bo[b, r, pl.ds(_D2 + k * L, L)] = hi
                return 0
            lax.fori_loop(0, _CT, _add, 0)
            pltpu.async_copy(bo.at[b], out_hbm.at[pl.ds(base + j * _CT, _CT)],
                             souts[b])

            @pl.when(j + 2 < nch)
            def _():
                _fire(j + 2, b)
        return 0
    lax.fori_loop(0, nch // 2, _pair, 0)

    for b in range(2):
        pltpu.make_async_copy(
            bo.at[b], out_hbm.at[pl.ds(base, _CT)], souts[b]).wait()


def kernel(x, W_route, b_route, W_noise, b_noise, W1, b1, W2, b2, noise):
    x2d = x.reshape(N, D)
    noise2d = noise.reshape(N, E)
    wcat = jnp.concatenate([W_route, W_noise], axis=1)
    bcat = jnp.tile(jnp.concatenate([b_route, b_noise])[None, :], (8, 1))

    t12, g1, xbf = _run_router(x2d, wcat, bcat, noise2d)
    idx, gate, loc1, loc2 = _dispatch(t12, g1)
    x_all = _gather_rows(xbf, idx)
    c = _run_ffn(x_all, W1, b1.reshape(E, 1, H), W2, b2.reshape(E, 1, D), gate)
    out2d = _combine(c, loc1, loc2)
    return out2d.reshape(B, S, D)


# speedup vs baseline: 3.6319x; 1.0224x over previous
import functools

import jax
import jax.numpy as jnp
from jax import lax
from jax.experimental import pallas as pl
from jax.experimental.pallas import tpu as pltpu
from jax.experimental.pallas import tpu_sc as plsc

B, S, D, E, K, H = 2, 4096, 2048, 8, 2, 512
N = B * S
CAP = N * K // E
EC = E * CAP
NC, NS, L = 2, 16, 16

_BR = 1024


_D2 = D // 2


def _pack16(v16):
    bits = lax.bitcast_convert_type(v16, jnp.uint16)
    lo = bits[:, :_D2].astype(jnp.int32)
    hi = bits[:, _D2:].astype(jnp.int32)
    return lo | (hi << 16)


def _unpack16(vi32):
    lo = vi32.astype(jnp.uint16)
    hi = lax.shift_right_logical(vi32, 16).astype(jnp.uint16)
    return jnp.concatenate(
        [lax.bitcast_convert_type(lo, jnp.bfloat16),
         lax.bitcast_convert_type(hi, jnp.bfloat16)], axis=1)


def _router_body(x_ref, w_ref, b_ref, noise_ref, t12_ref, g1_ref, xbf_ref):
    xbf_ref[...] = _pack16(x_ref[...].astype(jnp.bfloat16))
    lg = jnp.dot(x_ref[...], w_ref[...], preferred_element_type=jnp.float32)
    bias = b_ref[0:1, :]
    logits = lg[:, :E] + bias[:, :E]
    nlog = lg[:, E:] + bias[:, E:]
    sp = jnp.maximum(nlog, 0.0) + jnp.log1p(jnp.exp(-jnp.abs(nlog)))
    noisy = logits + noise_ref[...] * sp

    iota = lax.broadcasted_iota(jnp.int32, (_BR, E), 1)
    m1 = jnp.max(noisy, axis=1, keepdims=True)
    i1 = jnp.min(jnp.where(noisy == m1, iota, E), axis=1, keepdims=True)
    n2 = jnp.where(iota == i1, -jnp.inf, noisy)
    m2 = jnp.max(n2, axis=1, keepdims=True)
    i2 = jnp.min(jnp.where(n2 == m2, iota, E), axis=1, keepdims=True)
    r = jnp.exp(m2 - m1)
    g1 = 1.0 / (1.0 + r)
    t12_ref[...] = ((i1 << 3) | i2).reshape(_BR)
    g1_ref[...] = g1.reshape(_BR)


def _run_router(x2d, wcat, bcat, noise2d):
    return pl.pallas_call(
        _router_body,
        grid=(N // _BR,),
        in_specs=[
            pl.BlockSpec((_BR, D), lambda i: (i, 0)),
            pl.BlockSpec((D, 2 * E), lambda i: (0, 0)),
            pl.BlockSpec((8, 2 * E), lambda i: (0, 0)),
            pl.BlockSpec((_BR, E), lambda i: (i, 0)),
        ],
        out_specs=[
            pl.BlockSpec((_BR,), lambda i: (i,)),
            pl.BlockSpec((_BR,), lambda i: (i,)),
            pl.BlockSpec((_BR, _D2), lambda i: (i, 0)),
        ],
        out_shape=[
            jax.ShapeDtypeStruct((N,), jnp.int32),
            jax.ShapeDtypeStruct((N,), jnp.float32),
            jax.ShapeDtypeStruct((N, _D2), jnp.int32),
        ],
    )(x2d, wcat, bcat, noise2d)


_TT = N // NS
_TV = _TT // L
_SLOT_T = EC // NS

_disp_mesh = plsc.VectorSubcoreMesh(
    core_axis_name="c", subcore_axis_name="s", num_cores=1)


@functools.partial(
    pl.kernel,
    out_type=[
        jax.ShapeDtypeStruct((EC,), jnp.int32),
        jax.ShapeDtypeStruct((EC,), jnp.float32),
        jax.ShapeDtypeStruct((N,), jnp.int32),
        jax.ShapeDtypeStruct((N,), jnp.int32),
    ],
    mesh=_disp_mesh,
    scratch_types=[
        pltpu.VMEM((_TT,), jnp.int32),
        pltpu.VMEM((_TT,), jnp.float32),
        pltpu.VMEM((L,), jnp.int32),
        pltpu.VMEM((NS, L), jnp.int32),
        pltpu.VMEM((EC,), jnp.int32),
        pltpu.VMEM((EC,), jnp.float32),
        pltpu.VMEM((_TT,), jnp.int32),
        pltpu.VMEM((_TT,), jnp.int32),
        pltpu.VMEM((_SLOT_T,), jnp.int32),
        pltpu.VMEM((_SLOT_T,), jnp.float32),
        pltpu.VMEM((_SLOT_T,), jnp.int32),
        pltpu.VMEM((_SLOT_T,), jnp.float32),
        pltpu.VMEM_SHARED((NS, L), jnp.int32),
        pltpu.VMEM_SHARED((NS, EC), jnp.int32),
        pltpu.VMEM_SHARED((NS, EC), jnp.float32),
    ],
    compiler_params=pltpu.CompilerParams(needs_layout_passes=False),
)
def _dispatch(t12_hbm, g1_hbm, idx_hbm, gate_hbm, loc1_hbm, loc2_hbm,
              t12v, g1v, cntv, cnt_all, idxb, gateb, loc1b, loc2b,
              acc_i, acc_g, tmp_i, tmp_g, cnt_sh, idx_sh, gate_sh):
    sid = lax.axis_index("s")
    base = sid * _TT
    iota = lax.iota(jnp.int32, L)
    zero_i = jnp.zeros((L,), jnp.int32)
    zero_f = jnp.zeros((L,), jnp.float32)

    pltpu.sync_copy(t12_hbm.at[pl.ds(base, _TT)], t12v)
    pltpu.sync_copy(g1_hbm.at[pl.ds(base, _TT)], g1v)

    def _zero(i, _):
        idxb[pl.ds(i * L, L)] = zero_i
        gateb[pl.ds(i * L, L)] = zero_f
        return 0
    lax.fori_loop(0, EC // L, _zero, 0)

    def _count(i, cnt):
        t12x = t12v[pl.ds(i * L, L)]
        t1x = t12x >> 3
        t2x = t12x & 7
        for e in range(E):
            m = (t1x == e) | (t2x == e)
            c = jnp.sum(jnp.where(m, 1, 0))
            cnt = cnt + jnp.where(iota == e, c, 0)
        return cnt
    cnt = lax.fori_loop(0, _TV, _count, jnp.zeros((L,), jnp.int32))
    cntv[...] = cnt
    pltpu.sync_copy(cntv, cnt_sh.at[sid])
    plsc.subcore_barrier()
    pltpu.sync_copy(cnt_sh, cnt_all)

    offv = jnp.zeros((L,), jnp.int32)
    for w in range(NS):
        offv = offv + jnp.where(jnp.int32(w) < sid, cnt_all[w], 0)
    offs = [jnp.sum(jnp.where(iota == e, offv, 0)) for e in range(E)]

    def _scan(i, carry):
        offs = list(carry)
        sl = pl.ds(i * L, L)
        t12x = t12v[sl]
        t1x = t12x >> 3
        t2x = t12x & 7
        g1x = g1v[sl]
        g2x = 1.0 - g1x
        tok = base + i * L + iota
        l1 = jnp.full((L,), EC, jnp.int32)
        l2 = jnp.full((L,), EC, jnp.int32)
        for e in range(E):
            m1 = t1x == e
            m2 = t2x == e
            m = m1 | m2
            ones = jnp.where(m, 1, 0)
            cs = plsc.cumsum(ones)
            pos = offs[e] + cs - 1
            ok = m & (pos < CAP)
            dst = pos + e * CAP
            plsc.store_scatter(idxb, [dst], tok, mask=ok)
            gx = jnp.where(m1, g1x, g2x)
            plsc.store_scatter(gateb, [dst], gx, mask=ok)
            l1 = jnp.where(m1 & ok, dst, l1)
            l2 = jnp.where(m2 & ok, dst, l2)
            offs[e] = offs[e] + jnp.sum(ones)
        loc1b[sl] = l1
        loc2b[sl] = l2
        return tuple(offs)
    lax.fori_loop(0, _TV, _scan, tuple(offs))

    pltpu.sync_copy(loc1b, loc1_hbm.at[pl.ds(base, _TT)])
    pltpu.sync_copy(loc2b, loc2_hbm.at[pl.ds(base, _TT)])

    pltpu.sync_copy(idxb, idx_sh.at[sid])
    pltpu.sync_copy(gateb, gate_sh.at[sid])
    plsc.subcore_barrier()

    sbase = sid * _SLOT_T

    def _zacc(i, _):
        acc_i[pl.ds(i * L, L)] = zero_i
        acc_g[pl.ds(i * L, L)] = zero_f
        return 0
    lax.fori_loop(0, _SLOT_T // L, _zacc, 0)

    for w in range(NS):
        pltpu.sync_copy(idx_sh.at[w, pl.ds(sbase, _SLOT_T)], tmp_i)
        pltpu.sync_copy(gate_sh.at[w, pl.ds(sbase, _SLOT_T)], tmp_g)

        def _acc(i, _):
            sl = pl.ds(i * L, L)
            plsc.addupdate(acc_i.at[sl], tmp_i[sl])
            plsc.addupdate(acc_g.at[sl], tmp_g[sl])
            return 0
        lax.fori_loop(0, _SLOT_T // L, _acc, 0)

    pltpu.sync_copy(acc_i, idx_hbm.at[pl.ds(sbase, _SLOT_T)])
    pltpu.sync_copy(acc_g, gate_hbm.at[pl.ds(sbase, _SLOT_T)])


_GR = 32
_g_mesh = plsc.VectorSubcoreMesh(core_axis_name="c", subcore_axis_name="s")


def _make_gather(nrows):
    @functools.partial(
        pl.kernel,
        out_type=jax.ShapeDtypeStruct((nrows, _D2), jnp.int32),
        mesh=_g_mesh,
        scratch_types=[
            pltpu.VMEM((2, _GR), jnp.int32),
            pltpu.VMEM((2, _GR, _D2), jnp.int32),
            pltpu.SemaphoreType.DMA,
            pltpu.SemaphoreType.DMA,
        ],
        compiler_params=pltpu.CompilerParams(needs_layout_passes=False),
    )
    def _gather_rows(x_hbm, idx_hbm, out_hbm, idxv, rows, sem0, sem1):
        wid = lax.axis_index("s") * NC + lax.axis_index("c")
        rpt = nrows // (NC * NS)
        nch = rpt // _GR
        base = wid * rpt
        sems = (sem0, sem1)

        for b in range(2):
            pltpu.sync_copy(idx_hbm.at[pl.ds(base + b * _GR, _GR)], idxv.at[b])
            pltpu.async_copy(x_hbm.at[idxv.at[b]], rows.at[b], sems[b])

        def _pair(m, _):
            for b in range(2):
                j = 2 * m + b
                pltpu.make_async_copy(x_hbm.at[idxv.at[b]], rows.at[b],
                                      sems[b]).wait()
                pltpu.sync_copy(rows.at[b],
                                out_hbm.at[pl.ds(base + j * _GR, _GR)])

                @pl.when(j + 2 < nch)
                def _():
                    pltpu.sync_copy(
                        idx_hbm.at[pl.ds(base + (j + 2) * _GR, _GR)],
                        idxv.at[b])
                    pltpu.async_copy(x_hbm.at[idxv.at[b]], rows.at[b], sems[b])
            return 0
        lax.fori_loop(0, nch // 2, _pair, 0)

    return _gather_rows


_gather_half = _make_gather(EC // 2)


_BM = 512
_NM = CAP // _BM
_CPAD = EC + _NM * _BM


def _ffn_body(*refs, ne, has_alias):
    if has_alias:
        x_ref, w1_ref, b1_ref, w2_ref, b2_ref, g_ref, _cprev, c_ref = refs
    else:
        x_ref, w1_ref, b1_ref, w2_ref, b2_ref, g_ref, c_ref = refs
    e = pl.program_id(0)

    @pl.when(e < ne)
    def _():
        xb16 = _unpack16(x_ref[...])
        h = jnp.dot(xb16, w1_ref[0].astype(jnp.bfloat16),
                    preferred_element_type=jnp.float32) + b1_ref[0]
        h = 0.5 * h * (1.0 + lax.erf(h * 0.7071067811865476))
        o = jnp.dot(h.astype(jnp.bfloat16), w2_ref[0].astype(jnp.bfloat16),
                    preferred_element_type=jnp.float32) + b2_ref[0]
        g = g_ref[...].reshape(_BM, 1)
        c_ref[...] = _pack16(((xb16.astype(jnp.float32) + o) * g)
                             .astype(jnp.bfloat16))

    @pl.when(e == ne)
    def _():
        c_ref[...] = jnp.zeros((_BM, _D2), jnp.int32)


_EH = E // 2


def _run_ffn(x_half, w1, b1r, w2, b2r, gate_half, e0, npad, c_prev=None):
    ce = lambda e: jnp.minimum(e, _EH - 1)
    ins = [x_half, w1, b1r, w2, b2r, gate_half]
    in_specs = [
        pl.BlockSpec((_BM, _D2), lambda e, m: (ce(e) * _NM + m, 0)),
        pl.BlockSpec((1, D, H), lambda e, m: (ce(e) + e0, 0, 0)),
        pl.BlockSpec((1, 1, H), lambda e, m: (ce(e) + e0, 0, 0)),
        pl.BlockSpec((1, H, D), lambda e, m: (ce(e) + e0, 0, 0)),
        pl.BlockSpec((1, 1, D), lambda e, m: (ce(e) + e0, 0, 0)),
        pl.BlockSpec((_BM,), lambda e, m: (ce(e) * _NM + m,)),
    ]
    kwargs = {}
    if c_prev is not None:
        ins.append(c_prev)
        in_specs.append(pl.BlockSpec(memory_space=pltpu.MemorySpace.HBM))
        kwargs["input_output_aliases"] = {6: 0}
    return pl.pallas_call(
        functools.partial(_ffn_body, ne=_EH, has_alias=c_prev is not None),
        grid=(_EH + (1 if npad else 0), _NM),
        in_specs=in_specs,
        out_specs=pl.BlockSpec(
            (_BM, _D2), lambda e, m: ((e + e0) * _NM + m, 0)),
        out_shape=jax.ShapeDtypeStruct((_CPAD, _D2), jnp.int32),
        **kwargs,
    )(*ins)


_CT = 8
_MSK = jnp.int32(-65536)


@functools.partial(
    pl.kernel,
    out_type=jax.ShapeDtypeStruct((N, D), jnp.float32),
    mesh=_g_mesh,
    scratch_types=[
        pltpu.VMEM((2, _CT), jnp.int32),
        pltpu.VMEM((2, _CT), jnp.int32),
        pltpu.VMEM((2, _CT, _D2), jnp.int32),
        pltpu.VMEM((2, _CT, _D2), jnp.int32),
        pltpu.VMEM((2, _CT, D), jnp.float32),
        pltpu.SemaphoreType.DMA,
        pltpu.SemaphoreType.DMA,
        pltpu.SemaphoreType.DMA,
        pltpu.SemaphoreType.DMA,
    ],
    compiler_params=pltpu.CompilerParams(needs_layout_passes=False),
)
def _combine(c_hbm, loc1_hbm, loc2_hbm, out_hbm, l1v, l2v, b1, b2, bo,
             sem0, sem1, so0, so1):
    wid = lax.axis_index("s") * NC + lax.axis_index("c")
    tpt = N // (NC * NS)
    nch = tpt // _CT
    base = wid * tpt
    sems = (sem0, sem1)
    souts = (so0, so1)

    def _fire(j, b):
        pltpu.sync_copy(loc1_hbm.at[pl.ds(base + j * _CT, _CT)], l1v.at[b])
        pltpu.sync_copy(loc2_hbm.at[pl.ds(base + j * _CT, _CT)], l2v.at[b])
        pltpu.async_copy(c_hbm.at[l1v.at[b]], b1.at[b], sems[b])
        pltpu.async_copy(c_hbm.at[l2v.at[b]], b2.at[b], sems[b])

    for b in range(2):
        _fire(b, b)

    def _pair(m, _):
        for b in range(2):
            j = 2 * m + b
            pltpu.make_async_copy(c_hbm.at[l1v.at[b]], b1.at[b],
                                  sems[b]).wait()
            pltpu.make_async_copy(c_hbm.at[l2v.at[b]], b2.at[b],
                                  sems[b]).wait()

            @pl.when(m > 0)
            def _():
                pltpu.make_async_copy(
                    bo.at[b], out_hbm.at[pl.ds(base, _CT)], souts[b]).wait()

            def _add(r, _):
                for k in range(_D2 // L):
                    sl = pl.ds(k * L, L)
                    v1 = b1[b, r, sl]
                    v2 = b2[b, r, sl]
                    lo = (plsc.bitcast(v1 << 16, jnp.float32)
                          + plsc.bitcast(v2 << 16, jnp.float32))
                    hi = (plsc.bitcast(v1 & _MSK, jnp.float32)
                          + plsc.bitcast(v2 & _MSK, jnp.float32))
                    bo[b, r, sl] = lo
                    bo[b, r, pl.ds(_D2 + k * L, L)] = hi
                return 0
            lax.fori_loop(0, _CT, _add, 0)
            pltpu.async_copy(bo.at[b], out_hbm.at[pl.ds(base + j * _CT, _CT)],
                             souts[b])

            @pl.when(j + 2 < nch)
            def _():
                _fire(j + 2, b)
        return 0
    lax.fori_loop(0, nch // 2, _pair, 0)

    for b in range(2):
        pltpu.make_async_copy(
            bo.at[b], out_hbm.at[pl.ds(base, _CT)], souts[b]).wait()


def kernel(x, W_route, b_route, W_noise, b_noise, W1, b1, W2, b2, noise):
    x2d = x.reshape(N, D)
    noise2d = noise.reshape(N, E)
    wcat = jnp.concatenate([W_route, W_noise], axis=1)
    bcat = jnp.tile(jnp.concatenate([b_route, b_noise])[None, :], (8, 1))

    t12, g1, xbf = _run_router(x2d, wcat, bcat, noise2d)
    idx, gate, loc1, loc2 = _dispatch(t12, g1)
    b1r = b1.reshape(E, 1, H)
    b2r = b2.reshape(E, 1, D)
    hc = EC // 2
    x_a = _gather_half(xbf, idx[:hc])
    x_b = _gather_half(xbf, idx[hc:])
    c_a = _run_ffn(x_a, W1, b1r, W2, b2r, gate[:hc], 0, 0)
    c = _run_ffn(x_b, W1, b1r, W2, b2r, gate[hc:], _EH, 1, c_prev=c_a)
    out2d = _combine(c, loc1, loc2)
    return out2d.reshape(B, S, D)
